# Initial kernel scaffold; baseline (speedup 1.0000x reference)
import jax
import jax.numpy as jnp
from jax.experimental import pallas as pl


def kernel(query_feat, labels_map, logits, pool_nerve, pool_cell, pool_bg):
    raise NotImplementedError("write your pallas kernel here")


# speedup vs baseline: 11.4532x; 11.4532x over previous
import functools

import jax
import jax.numpy as jnp
from jax.experimental import pallas as pl
from jax.experimental.pallas import tpu as pltpu

_TAU = 0.07
_M = 64
_TOPP = 3
_B = 2
_C = 256
_H = 128
_W = 128
_HW = _H * _W
_P = 16384
_RANK0 = _P // 4
_PB = 2048
_NB = _P // _PB
_RB = 32
_BIG = 1 << 30


def _f32_to_key(x):
    i = jax.lax.bitcast_convert_type(x, jnp.int32)
    return jnp.where(i >= 0, i, ~(i & jnp.int32(0x7FFFFFFF)))


def _key_to_f32(k):
    bits = jnp.where(k >= 0, k, (~k) | jnp.int32(-2147483648))
    return jax.lax.bitcast_convert_type(bits, jnp.float32)


def _idx_kernel(labels_ref, idx_ref):
    lab = labels_ref[0]
    mask = (lab > 0).astype(jnp.int32)
    mask_f = mask.astype(jnp.float32)
    r = jax.lax.broadcasted_iota(jnp.int32, (_W, _W), 0)
    c = jax.lax.broadcasted_iota(jnp.int32, (_W, _W), 1)
    incl_u = (r <= c).astype(jnp.float32)
    strict_l = (c < r).astype(jnp.float32)
    cs_w = jnp.dot(mask_f, incl_u, preferred_element_type=jnp.float32)
    prev_rows = jnp.dot(strict_l, mask_f,
                        preferred_element_type=jnp.float32)
    row_off = jnp.sum(prev_rows, axis=1, keepdims=True)
    cs = (cs_w + row_off).astype(jnp.int32)
    flat_i = (jax.lax.broadcasted_iota(jnp.int32, (_H, _W), 0) * _W
              + jax.lax.broadcasted_iota(jnp.int32, (_H, _W), 1))
    j = jax.lax.broadcasted_iota(jnp.int32, (_M, _H, _W), 0) + 1
    cand = jnp.where((mask[None] > 0) & (cs[None] == j), flat_i[None], _BIG)
    idx = jnp.min(cand, axis=(1, 2))
    idx = jnp.where(idx >= _BIG, 0, idx)
    idx_ref[0, 0] = idx


def _compute_idxs(labels_map):
    return pl.pallas_call(
        _idx_kernel,
        grid=(_B,),
        in_specs=[pl.BlockSpec((1, _H, _W), lambda b: (b, 0, 0))],
        out_specs=pl.BlockSpec((1, 1, _M), lambda b: (b, 0, 0)),
        out_shape=jax.ShapeDtypeStruct((_B, 1, _M), jnp.int32),
    )(labels_map.astype(jnp.int32))


def _qgather_kernel(idx_ref, qf_ref, q_ref):
    b = pl.program_id(0)
    jblk = pl.program_id(1)
    idx = idx_ref[0, 0]
    col = (jax.lax.broadcasted_iota(jnp.int32, (_PB, _M), 0)
           + jblk * _PB)
    oh = (col == idx[None, :]).astype(jnp.float32)
    blk = qf_ref[0]
    part = jax.lax.dot_general(
        oh, blk, (((0,), (1,)), ((), ())),
        preferred_element_type=jnp.float32)
    @pl.when(jblk == 0)
    def _():
        q_ref[0] = jnp.zeros_like(q_ref[0])
    q_ref[0] += part


def _gather_queries(idxs, query_feat):
    qf = query_feat.reshape(_B, _C, _HW)
    return pl.pallas_call(
        _qgather_kernel,
        grid=(_B, _NB),
        in_specs=[
            pl.BlockSpec((1, 1, _M), lambda b, j: (b, 0, 0)),
            pl.BlockSpec((1, _C, _PB), lambda b, j: (b, 0, j)),
        ],
        out_specs=pl.BlockSpec((1, _M, _C), lambda b, j: (b, 0, 0)),
        out_shape=jax.ShapeDtypeStruct((_B, _M, _C), jnp.float32),
    )(idxs, qf)


def _sims_kern(q_ref, pn_ref, pc_ref, pb_ref, s_ref):
    q = q_ref[...]
    n = jnp.sqrt(jnp.sum(q * q, axis=1, keepdims=True))
    qn = q / jnp.maximum(n, 1e-12)
    for t, pref in enumerate((pn_ref, pc_ref, pb_ref)):
        blk = pref[...]
        n2 = jnp.sum(blk * blk, axis=0, keepdims=True)
        scale = 1.0 / jnp.maximum(jnp.sqrt(n2), 1e-12)
        s = jnp.dot(qn, blk, preferred_element_type=jnp.float32) * scale
        s_ref[t] = s


def _compute_sims(q_all, pool_nerve, pool_cell, pool_bg):
    pool_spec = pl.BlockSpec((_C, _PB), lambda j: (0, j))
    return pl.pallas_call(
        _sims_kern,
        grid=(_NB,),
        in_specs=[
            pl.BlockSpec((_B * _M, _C), lambda j: (0, 0)),
            pool_spec, pool_spec, pool_spec,
        ],
        out_specs=pl.BlockSpec((3, _B * _M, _PB), lambda j: (0, 0, j)),
        out_shape=jax.ShapeDtypeStruct((3, _B * _M, _P), jnp.float32),
    )(q_all, pool_nerve, pool_cell, pool_bg)


def _rank_select(key, k):
    rows = key.shape[0]
    prefix_u = jnp.zeros((rows, 1), jnp.int32)
    for bit in range(31, -1, -1):
        trial_u = prefix_u | jnp.int32(1 << bit) if bit < 31 else jnp.full(
            (rows, 1), jnp.int32(-2147483648))
        trial_s = trial_u ^ jnp.int32(-2147483648)
        cnt = jnp.sum((key >= trial_s).astype(jnp.int32), axis=1,
                      keepdims=True)
        prefix_u = jnp.where(cnt >= k + 1, trial_u, prefix_u)
    return _key_to_f32(prefix_u ^ jnp.int32(-2147483648))


def _select_kern(s_ref, val_ref, idx_ref):
    pos = s_ref[0]
    colid = jax.lax.broadcasted_iota(jnp.int32, pos.shape, 1)
    cur = pos
    vals, idxs = [], []
    for _ in range(_TOPP):
        m = jnp.max(cur, axis=1, keepdims=True)
        arg = jnp.min(jnp.where(cur == m, colid, _BIG), axis=1,
                      keepdims=True)
        vals.append(m)
        idxs.append(arg)
        cur = jnp.where(colid == arg, -jnp.inf, cur)
    negs = []
    for t in (1, 2):
        key = _f32_to_key(s_ref[t])
        negs.append(_rank_select(key, _RANK0))
        negs.append(_rank_select(key, _RANK0 + 1))
    val_ref[...] = jnp.concatenate(vals + negs + [jnp.zeros_like(vals[0])],
                                   axis=1)
    idx_ref[...] = jnp.concatenate(
        idxs + [jnp.zeros_like(idxs[0])] * 5, axis=1)


def _select(sims):
    nq = _B * _M
    return pl.pallas_call(
        _select_kern,
        grid=(nq // _RB,),
        in_specs=[pl.BlockSpec((3, _RB, _P), lambda r: (0, r, 0))],
        out_specs=[
            pl.BlockSpec((_RB, 8), lambda r: (r, 0)),
            pl.BlockSpec((_RB, 8), lambda r: (r, 0)),
        ],
        out_shape=[
            jax.ShapeDtypeStruct((nq, 8), jnp.float32),
            jax.ShapeDtypeStruct((nq, 8), jnp.int32),
        ],
    )(sims)


def _pgather_kern(idx_ref, pool_ref, g_ref):
    jblk = pl.program_id(0)
    nq = _B * _M
    idx = idx_ref[...]
    blk = pool_ref[...]
    col = (jax.lax.broadcasted_iota(jnp.int32, (_PB, nq), 0) + jblk * _PB)
    @pl.when(jblk == 0)
    def _():
        g_ref[...] = jnp.zeros_like(g_ref[...])
    for t in range(_TOPP):
        oh = (col == idx[None, :, t]).astype(jnp.float32)
        part = jax.lax.dot_general(
            oh, blk, (((0,), (1,)), ((), ())),
            preferred_element_type=jnp.float32)
        g_ref[t] += part


def _gather_protos(top_idx, pool_nerve):
    nq = _B * _M
    return pl.pallas_call(
        _pgather_kern,
        grid=(_NB,),
        in_specs=[
            pl.BlockSpec((nq, 8), lambda j: (0, 0)),
            pl.BlockSpec((_C, _PB), lambda j: (0, j)),
        ],
        out_specs=pl.BlockSpec((_TOPP, nq, _C), lambda j: (0, 0, 0)),
        out_shape=jax.ShapeDtypeStruct((_TOPP, nq, _C), jnp.float32),
    )(top_idx, pool_nerve)


def _final_kern(val_ref, g_ref, out_ref):
    v = val_ref[...]
    c1, c2, c3 = g_ref[0], g_ref[1], g_ref[2]
    n1 = jnp.maximum(jnp.sqrt(jnp.sum(c1 * c1, axis=1, keepdims=True)), 1e-12)
    n2 = jnp.maximum(jnp.sqrt(jnp.sum(c2 * c2, axis=1, keepdims=True)), 1e-12)
    n3 = jnp.maximum(jnp.sqrt(jnp.sum(c3 * c3, axis=1, keepdims=True)), 1e-12)
    d12 = jnp.sum(c1 * c2, axis=1, keepdims=True) / (n1 * n2)
    d13 = jnp.sum(c1 * c3, axis=1, keepdims=True) / (n1 * n3)
    d23 = jnp.sum(c2 * c3, axis=1, keepdims=True) / (n2 * n3)
    msq = (3.0 + 2.0 * (d12 + d13 + d23)) / 9.0
    mnorm = jnp.maximum(jnp.sqrt(msq), 1e-12)
    s_mean = (v[:, 0:1] + v[:, 1:2] + v[:, 2:3]) / 3.0
    l_pos = (s_mean / mnorm) / _TAU
    l_neg = v[:, 3:7] / _TAU
    z = jnp.concatenate([l_pos, l_neg], axis=1)
    mx = jnp.max(z, axis=1, keepdims=True)
    lse = mx + jnp.log(jnp.sum(jnp.exp(z - mx), axis=1, keepdims=True))
    out_ref[...] = jnp.mean(lse - l_pos, keepdims=True)


def _final(vals, gcols):
    nq = _B * _M
    return pl.pallas_call(
        _final_kern,
        in_specs=[
            pl.BlockSpec((nq, 8), lambda: (0, 0)),
            pl.BlockSpec((_TOPP, nq, _C), lambda: (0, 0, 0)),
        ],
        out_specs=pl.BlockSpec((1, 1), lambda: (0, 0)),
        out_shape=jax.ShapeDtypeStruct((1, 1), jnp.float32),
    )(vals, gcols)


@jax.jit
def kernel(query_feat, labels_map, logits, pool_nerve, pool_cell, pool_bg):
    del logits
    idxs = _compute_idxs(labels_map)
    q = _gather_queries(idxs, query_feat)
    q_all = q.reshape(_B * _M, _C)
    sims = _compute_sims(q_all, pool_nerve, pool_cell, pool_bg)
    vals, top_idx = _select(sims)
    gcols = _gather_protos(top_idx, pool_nerve)
    out = _final(vals, gcols)
    return out[0, 0]


# speedup vs baseline: 13.3912x; 1.1692x over previous
import jax
import jax.numpy as jnp
from jax.experimental import pallas as pl
from jax.experimental.pallas import tpu as pltpu

_TAU = 0.07
_M = 64
_TOPP = 3
_B = 2
_C = 256
_H = 128
_W = 128
_HW = _H * _W
_P = 16384
_RANK0 = _P // 4
_PB = 1024
_NB = _P // _PB
_RC = 64
_BIG = 1 << 30
_IMIN = -2147483648


def _f32_to_key(x):
    i = jax.lax.bitcast_convert_type(x, jnp.int32)
    return jnp.where(i >= 0, i, ~(i & jnp.int32(0x7FFFFFFF)))


def _key_to_f32(k):
    bits = jnp.where(k >= 0, k, (~k) | jnp.int32(_IMIN))
    return jax.lax.bitcast_convert_type(bits, jnp.float32)


def _rank_pair(key, k):
    rows = key.shape[0]
    prefix_u = jnp.zeros((rows, 1), jnp.int32)
    for bit in range(31, -1, -1):
        if bit == 31:
            trial_u = jnp.full((rows, 1), jnp.int32(_IMIN))
        else:
            trial_u = prefix_u | jnp.int32(1 << bit)
        trial_s = trial_u ^ jnp.int32(_IMIN)
        cnt = jnp.sum((key >= trial_s).astype(jnp.int32), axis=1,
                      keepdims=True)
        prefix_u = jnp.where(cnt >= k + 1, trial_u, prefix_u)
    key1 = prefix_u ^ jnp.int32(_IMIN)
    ge = key >= key1
    cnt1 = jnp.sum(ge.astype(jnp.int32), axis=1, keepdims=True)
    below_max = jnp.max(jnp.where(ge, jnp.int32(_IMIN), key), axis=1,
                        keepdims=True)
    key2 = jnp.where(cnt1 >= k + 2, key1, below_max)
    return _key_to_f32(key1), _key_to_f32(key2)


def _top3(pos):
    colid = jax.lax.broadcasted_iota(jnp.int32, pos.shape, 1)
    cur = pos
    vals, idxs = [], []
    for _ in range(_TOPP):
        m = jnp.max(cur, axis=1, keepdims=True)
        arg = jnp.min(jnp.where(cur == m, colid, _BIG), axis=1, keepdims=True)
        vals.append(m)
        idxs.append(arg)
        cur = jnp.where(colid == arg, -jnp.inf, cur)
    return vals, idxs


def _idx_kernel(labels_ref, idx_ref):
    lab = labels_ref[0]
    mask = (lab > 0).astype(jnp.int32)
    mask_f = mask.astype(jnp.float32)
    r = jax.lax.broadcasted_iota(jnp.int32, (_W, _W), 0)
    c = jax.lax.broadcasted_iota(jnp.int32, (_W, _W), 1)
    incl_u = (r <= c).astype(jnp.float32)
    strict_l = (c < r).astype(jnp.float32)
    cs_w = jnp.dot(mask_f, incl_u, preferred_element_type=jnp.float32)
    prev_rows = jnp.dot(strict_l, mask_f,
                        preferred_element_type=jnp.float32)
    row_off = jnp.sum(prev_rows, axis=1, keepdims=True)
    cs = (cs_w + row_off).astype(jnp.int32)
    flat_i = (jax.lax.broadcasted_iota(jnp.int32, (_H, _W), 0) * _W
              + jax.lax.broadcasted_iota(jnp.int32, (_H, _W), 1))
    j = jax.lax.broadcasted_iota(jnp.int32, (_M, _H, _W), 0) + 1
    cand = jnp.where((mask[None] > 0) & (cs[None] == j), flat_i[None], _BIG)
    idx = jnp.min(cand, axis=(1, 2))
    idx = jnp.where(idx >= _BIG, 0, idx)
    idx_ref[0, 0] = idx


def _compute_idxs(labels_map):
    return pl.pallas_call(
        _idx_kernel,
        grid=(_B,),
        in_specs=[pl.BlockSpec((1, _H, _W), lambda b: (b, 0, 0))],
        out_specs=pl.BlockSpec((1, 1, _M), lambda b: (b, 0, 0)),
        out_shape=jax.ShapeDtypeStruct((_B, 1, _M), jnp.int32),
    )(labels_map.astype(jnp.int32))


def _qgather_kernel(idx_ref, qf_ref, q_ref):
    jblk = pl.program_id(1)
    idx = idx_ref[0, 0]
    col = (jax.lax.broadcasted_iota(jnp.int32, (_PB, _M), 0)
           + jblk * _PB)
    oh = (col == idx[None, :]).astype(jnp.float32)
    blk = qf_ref[0]
    part = jax.lax.dot_general(
        oh, blk, (((0,), (1,)), ((), ())),
        preferred_element_type=jnp.float32)
    @pl.when(jblk == 0)
    def _():
        q_ref[0] = jnp.zeros_like(q_ref[0])
    q_ref[0] += part


def _gather_queries(idxs, query_feat):
    qf = query_feat.reshape(_B, _C, _HW)
    return pl.pallas_call(
        _qgather_kernel,
        grid=(_B, _NB),
        in_specs=[
            pl.BlockSpec((1, 1, _M), lambda b, j: (b, 0, 0)),
            pl.BlockSpec((1, _C, _PB), lambda b, j: (b, 0, j)),
        ],
        out_specs=pl.BlockSpec((1, _M, _C), lambda b, j: (b, 0, 0)),
        out_shape=jax.ShapeDtypeStruct((_B, _M, _C), jnp.float32),
    )(idxs, qf)


def _simsel_kern(q_ref, pn_ref, pc_ref, pb_ref, val_ref, idx_ref, s_scr):
    jblk = pl.program_id(0)
    q = q_ref[...]
    n = jnp.sqrt(jnp.sum(q * q, axis=1, keepdims=True))
    qn = q / jnp.maximum(n, 1e-12)
    for t, pref in enumerate((pn_ref, pc_ref, pb_ref)):
        blk = pref[...]
        n2 = jnp.sum(blk * blk, axis=0, keepdims=True)
        scale = 1.0 / jnp.maximum(jnp.sqrt(n2), 1e-12)
        s = jnp.dot(qn, blk, preferred_element_type=jnp.float32) * scale
        s_scr[t, :, pl.ds(jblk * _PB, _PB)] = s

    @pl.when(jblk == _NB - 1)
    def _():
        nq = _B * _M
        for rc in range(nq // _RC):
            lo, hi = rc * _RC, (rc + 1) * _RC
            vals, idxs = _top3(s_scr[0, lo:hi, :])
            negs = []
            for t in (1, 2):
                key = _f32_to_key(s_scr[t, lo:hi, :])
                v1, v2 = _rank_pair(key, _RANK0)
                negs += [v1, v2]
            pad = jnp.zeros_like(vals[0])
            val_ref[lo:hi, :] = jnp.concatenate(vals + negs + [pad], axis=1)
            idx_ref[lo:hi, :] = jnp.concatenate(
                idxs + [jnp.zeros_like(idxs[0])] * 5, axis=1)


def _sims_select(q_all, pool_nerve, pool_cell, pool_bg):
    nq = _B * _M
    pool_spec = pl.BlockSpec((_C, _PB), lambda j: (0, j))
    return pl.pallas_call(
        _simsel_kern,
        grid=(_NB,),
        in_specs=[
            pl.BlockSpec((nq, _C), lambda j: (0, 0)),
            pool_spec, pool_spec, pool_spec,
        ],
        out_specs=[
            pl.BlockSpec((nq, 8), lambda j: (0, 0)),
            pl.BlockSpec((nq, 8), lambda j: (0, 0)),
        ],
        out_shape=[
            jax.ShapeDtypeStruct((nq, 8), jnp.float32),
            jax.ShapeDtypeStruct((nq, 8), jnp.int32),
        ],
        scratch_shapes=[pltpu.VMEM((3, nq, _P), jnp.float32)],
    )(q_all, pool_nerve, pool_cell, pool_bg)


def _pgfinal_kern(idx_ref, val_ref, pool_ref, out_ref, g_scr):
    jblk = pl.program_id(0)
    nq = _B * _M
    idx = idx_ref[...]
    blk = pool_ref[...]
    col = (jax.lax.broadcasted_iota(jnp.int32, (_PB, nq), 0) + jblk * _PB)
    @pl.when(jblk == 0)
    def _():
        g_scr[...] = jnp.zeros_like(g_scr[...])
    for t in range(_TOPP):
        oh = (col == idx[None, :, t]).astype(jnp.float32)
        part = jax.lax.dot_general(
            oh, blk, (((0,), (1,)), ((), ())),
            preferred_element_type=jnp.float32)
        g_scr[t] += part

    @pl.when(jblk == _NB - 1)
    def _():
        v = val_ref[...]
        c1, c2, c3 = g_scr[0], g_scr[1], g_scr[2]
        n1 = jnp.maximum(jnp.sqrt(jnp.sum(c1 * c1, axis=1, keepdims=True)),
                         1e-12)
        n2 = jnp.maximum(jnp.sqrt(jnp.sum(c2 * c2, axis=1, keepdims=True)),
                         1e-12)
        n3 = jnp.maximum(jnp.sqrt(jnp.sum(c3 * c3, axis=1, keepdims=True)),
                         1e-12)
        d12 = jnp.sum(c1 * c2, axis=1, keepdims=True) / (n1 * n2)
        d13 = jnp.sum(c1 * c3, axis=1, keepdims=True) / (n1 * n3)
        d23 = jnp.sum(c2 * c3, axis=1, keepdims=True) / (n2 * n3)
        msq = (3.0 + 2.0 * (d12 + d13 + d23)) / 9.0
        mnorm = jnp.maximum(jnp.sqrt(msq), 1e-12)
        s_mean = (v[:, 0:1] + v[:, 1:2] + v[:, 2:3]) / 3.0
        l_pos = (s_mean / mnorm) / _TAU
        l_neg = v[:, 3:7] / _TAU
        z = jnp.concatenate([l_pos, l_neg], axis=1)
        mx = jnp.max(z, axis=1, keepdims=True)
        lse = mx + jnp.log(jnp.sum(jnp.exp(z - mx), axis=1, keepdims=True))
        out_ref[...] = jnp.mean(lse - l_pos, keepdims=True)


def _proto_final(top_idx, vals, pool_nerve):
    nq = _B * _M
    return pl.pallas_call(
        _pgfinal_kern,
        grid=(_NB,),
        in_specs=[
            pl.BlockSpec((nq, 8), lambda j: (0, 0)),
            pl.BlockSpec((nq, 8), lambda j: (0, 0)),
            pl.BlockSpec((_C, _PB), lambda j: (0, j)),
        ],
        out_specs=pl.BlockSpec((1, 1), lambda j: (0, 0)),
        out_shape=jax.ShapeDtypeStruct((1, 1), jnp.float32),
        scratch_shapes=[pltpu.VMEM((_TOPP, nq, _C), jnp.float32)],
    )(top_idx, vals, pool_nerve)


@jax.jit
def kernel(query_feat, labels_map, logits, pool_nerve, pool_cell, pool_bg):
    del logits
    idxs = _compute_idxs(labels_map)
    q = _gather_queries(idxs, query_feat)
    q_all = q.reshape(_B * _M, _C)
    vals, top_idx = _sims_select(q_all, pool_nerve, pool_cell, pool_bg)
    out = _proto_final(top_idx, vals, pool_nerve)
    return out[0, 0]


# speedup vs baseline: 13.5610x; 1.0127x over previous
import functools

import jax
import jax.numpy as jnp
from jax import lax
from jax.experimental import pallas as pl
from jax.experimental.pallas import tpu as pltpu
from jax.experimental.pallas import tpu_sc as plsc

_TAU = 0.07
_M = 64
_TOPP = 3
_B = 2
_C = 256
_H = 128
_W = 128
_HW = _H * _W
_P = 16384
_RANK0 = _P // 4
_PB = 1024
_NB = _P // _PB
_RC = 64
_BIG = 1 << 30
_IMIN = -2147483648


def _f32_to_key(x):
    i = jax.lax.bitcast_convert_type(x, jnp.int32)
    return jnp.where(i >= 0, i, ~(i & jnp.int32(0x7FFFFFFF)))


def _key_to_f32(k):
    bits = jnp.where(k >= 0, k, (~k) | jnp.int32(_IMIN))
    return jax.lax.bitcast_convert_type(bits, jnp.float32)


def _rank_pair(key, k):
    rows = key.shape[0]
    prefix_u = jnp.zeros((rows, 1), jnp.int32)
    for bit in range(31, -1, -1):
        if bit == 31:
            trial_u = jnp.full((rows, 1), jnp.int32(_IMIN))
        else:
            trial_u = prefix_u | jnp.int32(1 << bit)
        trial_s = trial_u ^ jnp.int32(_IMIN)
        cnt = jnp.sum((key >= trial_s).astype(jnp.int32), axis=1,
                      keepdims=True)
        prefix_u = jnp.where(cnt >= k + 1, trial_u, prefix_u)
    key1 = prefix_u ^ jnp.int32(_IMIN)
    ge = key >= key1
    cnt1 = jnp.sum(ge.astype(jnp.int32), axis=1, keepdims=True)
    below_max = jnp.max(jnp.where(ge, jnp.int32(_IMIN), key), axis=1,
                        keepdims=True)
    key2 = jnp.where(cnt1 >= k + 2, key1, below_max)
    return _key_to_f32(key1), _key_to_f32(key2)


def _top3(pos):
    colid = jax.lax.broadcasted_iota(jnp.int32, pos.shape, 1)
    cur = pos
    vals, idxs = [], []
    for _ in range(_TOPP):
        m = jnp.max(cur, axis=1, keepdims=True)
        arg = jnp.min(jnp.where(cur == m, colid, _BIG), axis=1, keepdims=True)
        vals.append(m)
        idxs.append(arg)
        cur = jnp.where(colid == arg, -jnp.inf, cur)
    return vals, idxs


def _idx_kernel(labels_ref, idx_ref):
    lab = labels_ref[0]
    mask = (lab > 0).astype(jnp.int32)
    mask_f = mask.astype(jnp.float32)
    r = jax.lax.broadcasted_iota(jnp.int32, (_W, _W), 0)
    c = jax.lax.broadcasted_iota(jnp.int32, (_W, _W), 1)
    incl_u = (r <= c).astype(jnp.float32)
    strict_l = (c < r).astype(jnp.float32)
    cs_w = jnp.dot(mask_f, incl_u, preferred_element_type=jnp.float32)
    prev_rows = jnp.dot(strict_l, mask_f,
                        preferred_element_type=jnp.float32)
    row_off = jnp.sum(prev_rows, axis=1, keepdims=True)
    cs = (cs_w + row_off).astype(jnp.int32)
    flat_i = (jax.lax.broadcasted_iota(jnp.int32, (_H, _W), 0) * _W
              + jax.lax.broadcasted_iota(jnp.int32, (_H, _W), 1))
    j = jax.lax.broadcasted_iota(jnp.int32, (_M, _H, _W), 0) + 1
    cand = jnp.where((mask[None] > 0) & (cs[None] == j), flat_i[None], _BIG)
    idx = jnp.min(cand, axis=(1, 2))
    idx = jnp.where(idx >= _BIG, 0, idx)
    idx_ref[0, 0] = idx


def _compute_idxs(labels_map):
    return pl.pallas_call(
        _idx_kernel,
        grid=(_B,),
        in_specs=[pl.BlockSpec((1, _H, _W), lambda b: (b, 0, 0))],
        out_specs=pl.BlockSpec((1, 1, _M), lambda b: (b, 0, 0)),
        out_shape=jax.ShapeDtypeStruct((_B, 1, _M), jnp.int32),
    )(labels_map.astype(jnp.int32))


_NW = 32
_QPW = (_B * _M) // _NW
_RPW = _QPW * _C
_NCHUNK = _RPW // 128


def _rowidx_kernel(idx_ref, ril_ref):
    nq = _B * _M
    idx = idx_ref[...]
    c = jax.lax.broadcasted_iota(jnp.int32, (nq, _C), 1)
    b = jax.lax.broadcasted_iota(jnp.int32, (nq, _C), 0) >> 6
    ril_ref[...] = (b * _C + c) * (_HW // 128) + (idx >> 7)


def _rowidx(idx_flat):
    nq = _B * _M
    return pl.pallas_call(
        _rowidx_kernel,
        in_specs=[pl.BlockSpec((nq, 1), lambda: (0, 0))],
        out_specs=pl.BlockSpec((nq, _C), lambda: (0, 0)),
        out_shape=jax.ShapeDtypeStruct((nq, _C), jnp.int32),
    )(idx_flat.reshape(nq, 1))


def _sc_qgather_kernel(qf_hbm, il_hbm, out_hbm, idx_list, rows, sem):
    wid = lax.axis_index("c") * 16 + lax.axis_index("s")
    pltpu.sync_copy(il_hbm.at[wid], idx_list)
    for h2 in range(_QPW // 2):
        handles = []
        for k in range(4):
            a = h2 * 4 + k
            dst = rows.at[k // 2, pl.ds((k % 2) * 128, 128)]
            handles.append(
                pltpu.async_copy(qf_hbm.at[idx_list.at[a]], dst, sem))
        for h in handles:
            h.wait()
        pltpu.sync_copy(rows, out_hbm.at[pl.ds(wid * _QPW + h2 * 2, 2)])


def _sc_gather_queries(row_idx, query_feat):
    qf128 = query_feat.reshape(-1, 128)
    il = row_idx.reshape(_NW, _NCHUNK, 128)
    mesh = plsc.VectorSubcoreMesh(core_axis_name="c", subcore_axis_name="s")
    k = functools.partial(
        pl.kernel, mesh=mesh,
        out_type=jax.ShapeDtypeStruct((_B * _M, _C, 128), jnp.float32),
        scratch_types=[
            pltpu.VMEM((_NCHUNK, 128), jnp.int32),
            pltpu.VMEM((2, _C, 128), jnp.float32),
            pltpu.SemaphoreType.DMA,
        ],
    )(_sc_qgather_kernel)
    return k(qf128, il)


def _lane_kernel(idx_ref, q16_ref, q_ref):
    li = idx_ref[...] & 127
    oh = (jax.lax.broadcasted_iota(jnp.int32, (_B * _M, 1, 128), 2)
          == li[:, :, None]).astype(jnp.float32)
    q_ref[...] = jnp.sum(q16_ref[...] * oh, axis=2)


def _lane_select(idx_flat, q16):
    nq = _B * _M
    cb = 128
    return pl.pallas_call(
        _lane_kernel,
        grid=(_C // cb,),
        in_specs=[
            pl.BlockSpec((nq, 1), lambda j: (0, 0)),
            pl.BlockSpec((nq, cb, 128), lambda j: (0, j, 0)),
        ],
        out_specs=pl.BlockSpec((nq, cb), lambda j: (0, j)),
        out_shape=jax.ShapeDtypeStruct((nq, _C), jnp.float32),
    )(idx_flat.reshape(nq, 1), q16)


def _qgather_kernel(idx_ref, qf_ref, q_ref):
    jblk = pl.program_id(1)
    idx = idx_ref[0, 0]
    col = (jax.lax.broadcasted_iota(jnp.int32, (_PB, _M), 0)
           + jblk * _PB)
    oh = (col == idx[None, :]).astype(jnp.float32)
    blk = qf_ref[0]
    part = jax.lax.dot_general(
        oh, blk, (((0,), (1,)), ((), ())),
        preferred_element_type=jnp.float32)
    @pl.when(jblk == 0)
    def _():
        q_ref[0] = jnp.zeros_like(q_ref[0])
    q_ref[0] += part


def _gather_queries(idxs, query_feat):
    qf = query_feat.reshape(_B, _C, _HW)
    return pl.pallas_call(
        _qgather_kernel,
        grid=(_B, _NB),
        in_specs=[
            pl.BlockSpec((1, 1, _M), lambda b, j: (b, 0, 0)),
            pl.BlockSpec((1, _C, _PB), lambda b, j: (b, 0, j)),
        ],
        out_specs=pl.BlockSpec((1, _M, _C), lambda b, j: (b, 0, 0)),
        out_shape=jax.ShapeDtypeStruct((_B, _M, _C), jnp.float32),
    )(idxs, qf)


def _simsel_kern(q_ref, pn_ref, pc_ref, pb_ref, val_ref, idx_ref, s_scr):
    jblk = pl.program_id(0)
    q = q_ref[...]
    n = jnp.sqrt(jnp.sum(q * q, axis=1, keepdims=True))
    qn = q / jnp.maximum(n, 1e-12)
    for t, pref in enumerate((pn_ref, pc_ref, pb_ref)):
        blk = pref[...]
        n2 = jnp.sum(blk * blk, axis=0, keepdims=True)
        scale = 1.0 / jnp.maximum(jnp.sqrt(n2), 1e-12)
        s = jnp.dot(qn, blk, preferred_element_type=jnp.float32) * scale
        s_scr[t, :, pl.ds(jblk * _PB, _PB)] = s

    @pl.when(jblk == _NB - 1)
    def _():
        nq = _B * _M
        for rc in range(nq // _RC):
            lo, hi = rc * _RC, (rc + 1) * _RC
            vals, idxs = _top3(s_scr[0, lo:hi, :])
            negs = []
            for t in (1, 2):
                key = _f32_to_key(s_scr[t, lo:hi, :])
                v1, v2 = _rank_pair(key, _RANK0)
                negs += [v1, v2]
            pad = jnp.zeros_like(vals[0])
            val_ref[lo:hi, :] = jnp.concatenate(vals + negs + [pad], axis=1)
            idx_ref[lo:hi, :] = jnp.concatenate(
                idxs + [jnp.zeros_like(idxs[0])] * 5, axis=1)


def _sims_select(q_all, pool_nerve, pool_cell, pool_bg):
    nq = _B * _M
    pool_spec = pl.BlockSpec((_C, _PB), lambda j: (0, j))
    return pl.pallas_call(
        _simsel_kern,
        grid=(_NB,),
        in_specs=[
            pl.BlockSpec((nq, _C), lambda j: (0, 0)),
            pool_spec, pool_spec, pool_spec,
        ],
        out_specs=[
            pl.BlockSpec((nq, 8), lambda j: (0, 0)),
            pl.BlockSpec((nq, 8), lambda j: (0, 0)),
        ],
        out_shape=[
            jax.ShapeDtypeStruct((nq, 8), jnp.float32),
            jax.ShapeDtypeStruct((nq, 8), jnp.int32),
        ],
        scratch_shapes=[pltpu.VMEM((3, nq, _P), jnp.float32)],
    )(q_all, pool_nerve, pool_cell, pool_bg)


def _pgfinal_kern(idx_ref, val_ref, pool_ref, out_ref, g_scr):
    jblk = pl.program_id(0)
    nq = _B * _M
    idx = idx_ref[...]
    blk = pool_ref[...]
    col = (jax.lax.broadcasted_iota(jnp.int32, (_PB, nq), 0) + jblk * _PB)
    @pl.when(jblk == 0)
    def _():
        g_scr[...] = jnp.zeros_like(g_scr[...])
    for t in range(_TOPP):
        oh = (col == idx[None, :, t]).astype(jnp.float32)
        part = jax.lax.dot_general(
            oh, blk, (((0,), (1,)), ((), ())),
            preferred_element_type=jnp.float32)
        g_scr[t] += part

    @pl.when(jblk == _NB - 1)
    def _():
        v = val_ref[...]
        c1, c2, c3 = g_scr[0], g_scr[1], g_scr[2]
        n1 = jnp.maximum(jnp.sqrt(jnp.sum(c1 * c1, axis=1, keepdims=True)),
                         1e-12)
        n2 = jnp.maximum(jnp.sqrt(jnp.sum(c2 * c2, axis=1, keepdims=True)),
                         1e-12)
        n3 = jnp.maximum(jnp.sqrt(jnp.sum(c3 * c3, axis=1, keepdims=True)),
                         1e-12)
        d12 = jnp.sum(c1 * c2, axis=1, keepdims=True) / (n1 * n2)
        d13 = jnp.sum(c1 * c3, axis=1, keepdims=True) / (n1 * n3)
        d23 = jnp.sum(c2 * c3, axis=1, keepdims=True) / (n2 * n3)
        msq = (3.0 + 2.0 * (d12 + d13 + d23)) / 9.0
        mnorm = jnp.maximum(jnp.sqrt(msq), 1e-12)
        s_mean = (v[:, 0:1] + v[:, 1:2] + v[:, 2:3]) / 3.0
        l_pos = (s_mean / mnorm) / _TAU
        l_neg = v[:, 3:7] / _TAU
        z = jnp.concatenate([l_pos, l_neg], axis=1)
        mx = jnp.max(z, axis=1, keepdims=True)
        lse = mx + jnp.log(jnp.sum(jnp.exp(z - mx), axis=1, keepdims=True))
        out_ref[...] = jnp.mean(lse - l_pos, keepdims=True)


def _proto_final(top_idx, vals, pool_nerve):
    nq = _B * _M
    return pl.pallas_call(
        _pgfinal_kern,
        grid=(_NB,),
        in_specs=[
            pl.BlockSpec((nq, 8), lambda j: (0, 0)),
            pl.BlockSpec((nq, 8), lambda j: (0, 0)),
            pl.BlockSpec((_C, _PB), lambda j: (0, j)),
        ],
        out_specs=pl.BlockSpec((1, 1), lambda j: (0, 0)),
        out_shape=jax.ShapeDtypeStruct((1, 1), jnp.float32),
        scratch_shapes=[pltpu.VMEM((_TOPP, nq, _C), jnp.float32)],
    )(top_idx, vals, pool_nerve)


@jax.jit
def kernel(query_feat, labels_map, logits, pool_nerve, pool_cell, pool_bg):
    del logits
    idxs = _compute_idxs(labels_map)
    idx_flat = idxs.reshape(_B * _M)
    row_idx = _rowidx(idx_flat)
    q16 = _sc_gather_queries(row_idx, query_feat)
    q_all = _lane_select(idx_flat, q16)
    vals, top_idx = _sims_select(q_all, pool_nerve, pool_cell, pool_bg)
    out = _proto_final(top_idx, vals, pool_nerve)
    return out[0, 0]


# speedup vs baseline: 13.5749x; 1.0010x over previous
import functools

import jax
import jax.numpy as jnp
from jax import lax
from jax.experimental import pallas as pl
from jax.experimental.pallas import tpu as pltpu
from jax.experimental.pallas import tpu_sc as plsc

_TAU = 0.07
_M = 64
_TOPP = 3
_B = 2
_C = 256
_H = 128
_W = 128
_HW = _H * _W
_P = 16384
_RANK0 = _P // 4
_PB = 1024
_NB = _P // _PB
_RC = 64
_BIG = 1 << 30
_IMIN = -2147483648


def _f32_to_key(x):
    i = jax.lax.bitcast_convert_type(x, jnp.int32)
    return jnp.where(i >= 0, i, ~(i & jnp.int32(0x7FFFFFFF)))


def _key_to_f32(k):
    bits = jnp.where(k >= 0, k, (~k) | jnp.int32(_IMIN))
    return jax.lax.bitcast_convert_type(bits, jnp.float32)


def _rank_pair(key, k):
    rows = key.shape[0]
    prefix_u = jnp.zeros((rows, 1), jnp.int32)
    for bit in range(31, -1, -1):
        if bit == 31:
            trial_u = jnp.full((rows, 1), jnp.int32(_IMIN))
        else:
            trial_u = prefix_u | jnp.int32(1 << bit)
        trial_s = trial_u ^ jnp.int32(_IMIN)
        cnt = jnp.sum((key >= trial_s).astype(jnp.int32), axis=1,
                      keepdims=True)
        prefix_u = jnp.where(cnt >= k + 1, trial_u, prefix_u)
    key1 = prefix_u ^ jnp.int32(_IMIN)
    ge = key >= key1
    cnt1 = jnp.sum(ge.astype(jnp.int32), axis=1, keepdims=True)
    below_max = jnp.max(jnp.where(ge, jnp.int32(_IMIN), key), axis=1,
                        keepdims=True)
    key2 = jnp.where(cnt1 >= k + 2, key1, below_max)
    return _key_to_f32(key1), _key_to_f32(key2)


def _top3(pos):
    colid = jax.lax.broadcasted_iota(jnp.int32, pos.shape, 1)
    cur = pos
    vals, idxs = [], []
    for _ in range(_TOPP):
        m = jnp.max(cur, axis=1, keepdims=True)
        arg = jnp.min(jnp.where(cur == m, colid, _BIG), axis=1, keepdims=True)
        vals.append(m)
        idxs.append(arg)
        cur = jnp.where(colid == arg, -jnp.inf, cur)
    return vals, idxs


def _idx_kernel(labels_ref, idx_ref):
    lab = labels_ref[0]
    mask = (lab > 0).astype(jnp.int32)
    mask_f = mask.astype(jnp.float32)
    r = jax.lax.broadcasted_iota(jnp.int32, (_W, _W), 0)
    c = jax.lax.broadcasted_iota(jnp.int32, (_W, _W), 1)
    incl_u = (r <= c).astype(jnp.float32)
    strict_l = (c < r).astype(jnp.float32)
    cs_w = jnp.dot(mask_f, incl_u, preferred_element_type=jnp.float32)
    prev_rows = jnp.dot(strict_l, mask_f,
                        preferred_element_type=jnp.float32)
    row_off = jnp.sum(prev_rows, axis=1, keepdims=True)
    cs = (cs_w + row_off).astype(jnp.int32)
    flat_i = (jax.lax.broadcasted_iota(jnp.int32, (_H, _W), 0) * _W
              + jax.lax.broadcasted_iota(jnp.int32, (_H, _W), 1))
    j = jax.lax.broadcasted_iota(jnp.int32, (_M, _H, _W), 0) + 1
    cand = jnp.where((mask[None] > 0) & (cs[None] == j), flat_i[None], _BIG)
    idx = jnp.min(cand, axis=(1, 2))
    idx = jnp.where(idx >= _BIG, 0, idx)
    idx_ref[0, 0] = idx


def _compute_idxs(labels_map):
    return pl.pallas_call(
        _idx_kernel,
        grid=(_B,),
        in_specs=[pl.BlockSpec((1, _H, _W), lambda b: (b, 0, 0))],
        out_specs=pl.BlockSpec((1, 1, _M), lambda b: (b, 0, 0)),
        out_shape=jax.ShapeDtypeStruct((_B, 1, _M), jnp.int32),
    )(labels_map.astype(jnp.int32))


_NW = 32
_QPW = (_B * _M) // _NW
_RPW = _QPW * _C
_NCHUNK = _RPW // 128


def _rowidx_kernel(idx_ref, ril_ref):
    nq = _B * _M
    idx = idx_ref[...]
    c = jax.lax.broadcasted_iota(jnp.int32, (nq, _C), 1)
    b = jax.lax.broadcasted_iota(jnp.int32, (nq, _C), 0) >> 6
    ril_ref[...] = (b * _C + c) * (_HW // 128) + (idx >> 7)


def _rowidx(idx_flat):
    nq = _B * _M
    return pl.pallas_call(
        _rowidx_kernel,
        in_specs=[pl.BlockSpec((nq, 1), lambda: (0, 0))],
        out_specs=pl.BlockSpec((nq, _C), lambda: (0, 0)),
        out_shape=jax.ShapeDtypeStruct((nq, _C), jnp.int32),
    )(idx_flat.reshape(nq, 1))


def _sc_qgather_kernel(qf_hbm, il_hbm, out_hbm, idx_list, rows, sem):
    wid = lax.axis_index("c") * 16 + lax.axis_index("s")
    pltpu.sync_copy(il_hbm.at[wid], idx_list)
    for h2 in range(_QPW // 2):
        handles = []
        for k in range(4):
            a = h2 * 4 + k
            dst = rows.at[k // 2, pl.ds((k % 2) * 128, 128)]
            handles.append(
                pltpu.async_copy(qf_hbm.at[idx_list.at[a]], dst, sem))
        for h in handles:
            h.wait()
        pltpu.sync_copy(rows, out_hbm.at[pl.ds(wid * _QPW + h2 * 2, 2)])


def _sc_gather_queries(row_idx, query_feat):
    qf128 = query_feat.reshape(-1, 128)
    il = row_idx.reshape(_NW, _NCHUNK, 128)
    mesh = plsc.VectorSubcoreMesh(core_axis_name="c", subcore_axis_name="s")
    k = functools.partial(
        pl.kernel, mesh=mesh,
        out_type=jax.ShapeDtypeStruct((_B * _M, _C, 128), jnp.float32),
        scratch_types=[
            pltpu.VMEM((_NCHUNK, 128), jnp.int32),
            pltpu.VMEM((2, _C, 128), jnp.float32),
            pltpu.SemaphoreType.DMA,
        ],
    )(_sc_qgather_kernel)
    return k(qf128, il)


def _lane_kernel(idx_ref, q16_ref, q_ref):
    li = idx_ref[...] & 127
    oh = (jax.lax.broadcasted_iota(jnp.int32, (_B * _M, 1, 128), 2)
          == li[:, :, None]).astype(jnp.float32)
    q_ref[...] = jnp.sum(q16_ref[...] * oh, axis=2)


def _lane_select(idx_flat, q16):
    nq = _B * _M
    cb = 128
    return pl.pallas_call(
        _lane_kernel,
        grid=(_C // cb,),
        in_specs=[
            pl.BlockSpec((nq, 1), lambda j: (0, 0)),
            pl.BlockSpec((nq, cb, 128), lambda j: (0, j, 0)),
        ],
        out_specs=pl.BlockSpec((nq, cb), lambda j: (0, j)),
        out_shape=jax.ShapeDtypeStruct((nq, _C), jnp.float32),
    )(idx_flat.reshape(nq, 1), q16)


def _simsel_kern(q_ref, pn_ref, pc_ref, pb_ref, val_ref, idx_ref, s_scr):
    jblk = pl.program_id(0)
    q = q_ref[...]
    n = jnp.sqrt(jnp.sum(q * q, axis=1, keepdims=True))
    qn = q / jnp.maximum(n, 1e-12)
    for t, pref in enumerate((pn_ref, pc_ref, pb_ref)):
        blk = pref[...]
        n2 = jnp.sum(blk * blk, axis=0, keepdims=True)
        scale = 1.0 / jnp.maximum(jnp.sqrt(n2), 1e-12)
        s = jnp.dot(qn, blk, preferred_element_type=jnp.float32) * scale
        s_scr[t, :, pl.ds(jblk * _PB, _PB)] = s

    @pl.when(jblk == _NB - 1)
    def _():
        nq = _B * _M
        for rc in range(nq // _RC):
            lo, hi = rc * _RC, (rc + 1) * _RC
            vals, idxs = _top3(s_scr[0, lo:hi, :])
            negs = []
            for t in (1, 2):
                key = _f32_to_key(s_scr[t, lo:hi, :])
                v1, v2 = _rank_pair(key, _RANK0)
                negs += [v1, v2]
            pad = jnp.zeros_like(vals[0])
            val_ref[lo:hi, :] = jnp.concatenate(vals + negs + [pad], axis=1)
            idx_ref[lo:hi, :] = jnp.concatenate(
                idxs + [jnp.zeros_like(idxs[0])] * 5, axis=1)


def _sims_select(q_all, pool_nerve, pool_cell, pool_bg):
    nq = _B * _M
    pool_spec = pl.BlockSpec((_C, _PB), lambda j: (0, j))
    return pl.pallas_call(
        _simsel_kern,
        grid=(_NB,),
        in_specs=[
            pl.BlockSpec((nq, _C), lambda j: (0, 0)),
            pool_spec, pool_spec, pool_spec,
        ],
        out_specs=[
            pl.BlockSpec((nq, 8), lambda j: (0, 0)),
            pl.BlockSpec((nq, 8), lambda j: (0, 0)),
        ],
        out_shape=[
            jax.ShapeDtypeStruct((nq, 8), jnp.float32),
            jax.ShapeDtypeStruct((nq, 8), jnp.int32),
        ],
        scratch_shapes=[pltpu.VMEM((3, nq, _P), jnp.float32)],
    )(q_all, pool_nerve, pool_cell, pool_bg)


def _pgfinal_kern(idx_ref, val_ref, pool_ref, out_ref, g_scr):
    jblk = pl.program_id(0)
    nq = _B * _M
    idx = idx_ref[...]
    blk = pool_ref[...]
    col = (jax.lax.broadcasted_iota(jnp.int32, (_PB, nq), 0) + jblk * _PB)
    @pl.when(jblk == 0)
    def _():
        g_scr[...] = jnp.zeros_like(g_scr[...])
    for t in range(_TOPP):
        oh = (col == idx[None, :, t]).astype(jnp.float32)
        part = jax.lax.dot_general(
            oh, blk, (((0,), (1,)), ((), ())),
            preferred_element_type=jnp.float32)
        g_scr[t] += part

    @pl.when(jblk == _NB - 1)
    def _():
        v = val_ref[...]
        c1, c2, c3 = g_scr[0], g_scr[1], g_scr[2]
        n1 = jnp.maximum(jnp.sqrt(jnp.sum(c1 * c1, axis=1, keepdims=True)),
                         1e-12)
        n2 = jnp.maximum(jnp.sqrt(jnp.sum(c2 * c2, axis=1, keepdims=True)),
                         1e-12)
        n3 = jnp.maximum(jnp.sqrt(jnp.sum(c3 * c3, axis=1, keepdims=True)),
                         1e-12)
        d12 = jnp.sum(c1 * c2, axis=1, keepdims=True) / (n1 * n2)
        d13 = jnp.sum(c1 * c3, axis=1, keepdims=True) / (n1 * n3)
        d23 = jnp.sum(c2 * c3, axis=1, keepdims=True) / (n2 * n3)
        msq = (3.0 + 2.0 * (d12 + d13 + d23)) / 9.0
        mnorm = jnp.maximum(jnp.sqrt(msq), 1e-12)
        s_mean = (v[:, 0:1] + v[:, 1:2] + v[:, 2:3]) / 3.0
        l_pos = (s_mean / mnorm) / _TAU
        l_neg = v[:, 3:7] / _TAU
        z = jnp.concatenate([l_pos, l_neg], axis=1)
        mx = jnp.max(z, axis=1, keepdims=True)
        lse = mx + jnp.log(jnp.sum(jnp.exp(z - mx), axis=1, keepdims=True))
        out_ref[...] = jnp.mean(lse - l_pos, keepdims=True)


def _proto_final(top_idx, vals, pool_nerve):
    nq = _B * _M
    return pl.pallas_call(
        _pgfinal_kern,
        grid=(_NB,),
        in_specs=[
            pl.BlockSpec((nq, 8), lambda j: (0, 0)),
            pl.BlockSpec((nq, 8), lambda j: (0, 0)),
            pl.BlockSpec((_C, _PB), lambda j: (0, j)),
        ],
        out_specs=pl.BlockSpec((1, 1), lambda j: (0, 0)),
        out_shape=jax.ShapeDtypeStruct((1, 1), jnp.float32),
        scratch_shapes=[pltpu.VMEM((_TOPP, nq, _C), jnp.float32)],
    )(top_idx, vals, pool_nerve)


@jax.jit
def kernel(query_feat, labels_map, logits, pool_nerve, pool_cell, pool_bg):
    del logits
    idxs = _compute_idxs(labels_map)
    idx_flat = idxs.reshape(_B * _M)
    row_idx = _rowidx(idx_flat)
    q16 = _sc_gather_queries(row_idx, query_feat)
    q_all = _lane_select(idx_flat, q16)
    vals, top_idx = _sims_select(q_all, pool_nerve, pool_cell, pool_bg)
    out = _proto_final(top_idx, vals, pool_nerve)
    return out[0, 0]
